# baseline (speedup 1.0000x reference)
import functools

import jax
import jax.numpy as jnp
from jax import lax
from jax.experimental import pallas as pl
from jax.experimental.pallas import tpu as pltpu
from jax.experimental.pallas import tpu_sc as plsc

N = 10000
NP = 10240
H = 4
C = 32
HC = 128
G = 64
AW = 48
NC, NS = 2, 16
NW = NC * NS
KE = 128
GP = 2
GE = GP * KE
CH = 84
NG = CH // GP
EPT = KE * CH
EPAD = EPT * NW
RB = 512
GRID = NP // RB
RPT = NP // NS
ZR = 64


def _edge_body(xl0, xl1, xl2, xl3, xr0, xr1, xr2, xr3, src_hbm, dst_hbm,
               att_hbm, out_hbm, src_v, dst_v, xl_a, xl_b, xr_a, xr_b,
               ctr_a, ctr_b, att_v, zrow, acc, sem_a, sem_b, sem_sa, sem_sb):
    cid = lax.axis_index("c")
    sid = lax.axis_index("s")
    wid = sid * NC + cid
    xls = (xl0, xl1, xl2, xl3)
    xrs = (xr0, xr1, xr2, xr3)

    def zr_init(i, carry):
        for j in range(AW // 16):
            zrow[i, pl.ds(16 * j, 16)] = jnp.zeros((16,), jnp.float32)
        return carry
    lax.fori_loop(0, ZR, zr_init, 0)

    def zero_acc(t, carry):
        pltpu.sync_copy(zrow, acc.at[pl.ds(sid * RPT + t * ZR, ZR)])
        return carry

    pltpu.sync_copy(src_hbm.at[wid], src_v)
    pltpu.sync_copy(dst_hbm.at[wid], dst_v)
    pltpu.sync_copy(att_hbm, att_v)
    lax.fori_loop(0, RPT // ZR, zero_acc, 0)
    plsc.subcore_barrier()

    lane = lax.iota(jnp.int32, 16)
    _dn = lax.GatherDimensionNumbers(offset_dims=(), collapsed_slice_dims=(0,),
                                     start_index_map=(0,))

    def _lanperm(vec, idx):
        return lax.gather(vec, idx.reshape(16, 1), _dn, (1,),
                          mode=lax.GatherScatterMode.PROMISE_IN_BOUNDS)

    for h in range(H):
        def fire_gather(g, xlb, xrb, sem, h=h):
            for j in range(GP):
                ch = GP * g + j
                pltpu.async_copy(xls[h].at[src_v.at[ch]],
                                 xlb.at[pl.ds(KE * j, KE)], sem)
                pltpu.async_copy(xrs[h].at[dst_v.at[ch]],
                                 xrb.at[pl.ds(KE * j, KE)], sem)

        def wait_gather(g, xlb, xrb, sem, h=h):
            for j in range(GP):
                ch = GP * g + j
                pltpu.make_async_copy(xls[h].at[src_v.at[ch]],
                                      xlb.at[pl.ds(KE * j, KE)], sem).wait()
                pltpu.make_async_copy(xrs[h].at[dst_v.at[ch]],
                                      xrb.at[pl.ds(KE * j, KE)], sem).wait()

        def fire_scatter(g, ctr, sem):
            for j in range(GP):
                pltpu.async_copy(ctr.at[pl.ds(KE * j, KE)],
                                 acc.at[dst_v.at[GP * g + j]], sem, add=True)

        def wait_scatter(g, ctr, sem):
            for j in range(GP):
                pltpu.make_async_copy(ctr.at[pl.ds(KE * j, KE)],
                                      acc.at[dst_v.at[GP * g + j]], sem).wait()

        def compute(xlb, xrb, ctr, h=h):
            att01 = (att_v[pl.ds(32 * h, 16)], att_v[pl.ds(32 * h + 16, 16)])

            @plsc.parallel_loop(0, GE, unroll=8, carry=att01)
            def _(e, att):
                at0, at1 = att
                a0 = xlb[e, pl.ds(0, 16)]
                a1 = xlb[e, pl.ds(16, 16)]
                b0 = xrb[e, pl.ds(0, 16)]
                b1 = xrb[e, pl.ds(16, 16)]
                v0 = a0 + b0
                v1 = a1 + b1
                l0 = jnp.where(v0 > 0, v0, 0.2 * v0)
                l1 = jnp.where(v1 > 0, v1, 0.2 * v1)
                t = l0 * at0 + l1 * at1
                for sh in (8, 4, 2, 1):
                    t = t + _lanperm(t, jnp.bitwise_xor(lane, sh))
                u = jnp.exp(t)
                ctr[e, pl.ds(0, 16)] = u * a0
                ctr[e, pl.ds(16, 16)] = u * a1
                ctr[e, pl.ds(32, 16)] = u
                return att

        fire_gather(0, xl_a, xr_a, sem_a)
        fire_gather(1, xl_b, xr_b, sem_b)
        wait_gather(0, xl_a, xr_a, sem_a)
        compute(xl_a, xr_a, ctr_a)
        fire_scatter(0, ctr_a, sem_sa)
        fire_gather(2, xl_a, xr_a, sem_a)
        wait_gather(1, xl_b, xr_b, sem_b)
        compute(xl_b, xr_b, ctr_b)
        fire_scatter(1, ctr_b, sem_sb)
        fire_gather(3, xl_b, xr_b, sem_b)

        def pair(k, carry):
            g0 = 2 * k + 2
            g1 = 2 * k + 3
            wait_gather(g0, xl_a, xr_a, sem_a)
            wait_scatter(g0 - 2, ctr_a, sem_sa)
            compute(xl_a, xr_a, ctr_a)
            fire_scatter(g0, ctr_a, sem_sa)
            fire_gather(g0 + 2, xl_a, xr_a, sem_a)
            wait_gather(g1, xl_b, xr_b, sem_b)
            wait_scatter(g1 - 2, ctr_b, sem_sb)
            compute(xl_b, xr_b, ctr_b)
            fire_scatter(g1, ctr_b, sem_sb)
            fire_gather(g1 + 2, xl_b, xr_b, sem_b)
            return carry

        lax.fori_loop(0, (NG - 4) // 2, pair, 0)

        g0 = NG - 2
        g1 = NG - 1
        wait_gather(g0, xl_a, xr_a, sem_a)
        wait_scatter(g0 - 2, ctr_a, sem_sa)
        compute(xl_a, xr_a, ctr_a)
        fire_scatter(g0, ctr_a, sem_sa)
        wait_gather(g1, xl_b, xr_b, sem_b)
        wait_scatter(g1 - 2, ctr_b, sem_sb)
        compute(xl_b, xr_b, ctr_b)
        fire_scatter(g1, ctr_b, sem_sb)
        wait_scatter(g0, ctr_a, sem_sa)
        wait_scatter(g1, ctr_b, sem_sb)

        plsc.subcore_barrier()
        pltpu.sync_copy(acc.at[pl.ds(sid * RPT, RPT)],
                        out_hbm.at[h, cid, pl.ds(sid * RPT, RPT)])
        lax.fori_loop(0, RPT // ZR, zero_acc, 0)
        plsc.subcore_barrier()


@functools.cache
def _make_edge_kernel():
  tab = jax.ShapeDtypeStruct((NP, C), jnp.float32)
  return pl.kernel(
    _edge_body,
    out_type=jax.ShapeDtypeStruct((H, NC, NP, AW), jnp.float32),
    mesh=plsc.VectorSubcoreMesh(core_axis_name="c", subcore_axis_name="s",
                                num_cores=NC, num_subcores=NS),
    compiler_params=pltpu.CompilerParams(use_tc_tiling_on_sc=False),
    scratch_types=[
        pltpu.VMEM((CH, KE), jnp.int32),
        pltpu.VMEM((CH, KE), jnp.int32),
        pltpu.VMEM((GE, C), jnp.float32),
        pltpu.VMEM((GE, C), jnp.float32),
        pltpu.VMEM((GE, C), jnp.float32),
        pltpu.VMEM((GE, C), jnp.float32),
        pltpu.VMEM((GE, AW), jnp.float32),
        pltpu.VMEM((GE, AW), jnp.float32),
        pltpu.VMEM((HC,), jnp.float32),
        pltpu.VMEM((ZR, AW), jnp.float32),
        pltpu.VMEM_SHARED((NP, AW), jnp.float32),
        pltpu.SemaphoreType.DMA,
        pltpu.SemaphoreType.DMA,
        pltpu.SemaphoreType.DMA,
        pltpu.SemaphoreType.DMA,
    ],
  )


def _lstm(y, h_prev, c_prev, wiht, whht, bcomb):
    gates = (jnp.dot(y, wiht, preferred_element_type=jnp.float32)
             + jnp.dot(h_prev, whht, preferred_element_type=jnp.float32)
             + bcomb)
    i = jax.nn.sigmoid(gates[:, 0:128])
    f = jax.nn.sigmoid(gates[:, 128:256])
    g = jnp.tanh(gates[:, 256:384])
    o = jax.nn.sigmoid(gates[:, 384:512])
    c2 = f * c_prev + i * g
    h2 = o * jnp.tanh(c2)
    return h2, c2


def _layernorm(y, g, b):
    m = jnp.mean(y, axis=-1, keepdims=True)
    v = jnp.mean((y - m) * (y - m), axis=-1, keepdims=True)
    return (y - m) / jnp.sqrt(v + 1e-5) * g + b


def _norm_ln_elu(part_ref, bias, ln_g, ln_b):
    ys = []
    for h in range(H):
        a = part_ref[h, 0] + part_ref[h, 1]
        den = jnp.maximum(a[:, 32:33], 1e-16)
        ys.append(a[:, 0:32] / den)
    y = jnp.concatenate(ys, axis=1) + bias
    y = _layernorm(y, ln_g, ln_b)
    return jnp.where(y > 0, y, jnp.exp(y) - 1.0)


def _split_heads(xfull, outs):
    for j in range(H):
        outs[j][...] = xfull[:, 32 * j:32 * j + 32]


def _stage_a(x_ref, w0t, b0, wih0t, bc0, wlt, wrt, h_o, c_o, *lr_o):
    xb = x_ref[...]
    x0 = jnp.dot(xb, w0t[...], preferred_element_type=jnp.float32) + b0[...]
    gates = jnp.dot(x0, wih0t[...], preferred_element_type=jnp.float32) + bc0[...]
    i = jax.nn.sigmoid(gates[:, 0:128])
    g = jnp.tanh(gates[:, 256:384])
    o = jax.nn.sigmoid(gates[:, 384:512])
    c2 = i * g
    h_o[...] = o * jnp.tanh(c2)
    c_o[...] = c2
    _split_heads(jnp.dot(xb, wlt[...], preferred_element_type=jnp.float32),
                 lr_o[0:H])
    _split_heads(jnp.dot(xb, wrt[...], preferred_element_type=jnp.float32),
                 lr_o[H:2 * H])


def _stage_b(part_ref, bias, ln_g, ln_b, h_ref, c_ref, wiht, whht, bcomb,
             wlt, wrt, h_o, c_o, *lr_o):
    y = _norm_ln_elu(part_ref, bias[...], ln_g[...], ln_b[...])
    h2, c2 = _lstm(y, h_ref[...], c_ref[...], wiht[...], whht[...], bcomb[...])
    h_o[...] = h2
    c_o[...] = c2
    _split_heads(jnp.dot(h2, wlt[...], preferred_element_type=jnp.float32),
                 lr_o[0:H])
    _split_heads(jnp.dot(h2, wrt[...], preferred_element_type=jnp.float32),
                 lr_o[H:2 * H])


def _stage_c(part_ref, bias, ln_g, ln_b, h_ref, c_ref, wiht, whht, bcomb,
             batch_ref, sums_o, cnt_o):
    pid = pl.program_id(0)

    @pl.when(pid == 0)
    def _():
        sums_o[...] = jnp.zeros_like(sums_o)
        cnt_o[...] = jnp.zeros_like(cnt_o)

    y = _norm_ln_elu(part_ref, bias[...], ln_g[...], ln_b[...])
    h2, _ = _lstm(y, h_ref[...], c_ref[...], wiht[...], whht[...], bcomb[...])
    onehot = (batch_ref[...] == lax.broadcasted_iota(jnp.int32, (1, G), 1))
    onehot = onehot.astype(jnp.float32)
    sums_o[...] += lax.dot_general(onehot, h2, (((0,), (0,)), ((), ())),
                                   preferred_element_type=jnp.float32)
    cnt = jnp.sum(onehot, axis=0)
    cnt_o[...] += jnp.broadcast_to(cnt.reshape(G, 1), (G, HC))


def _final(sums_ref, cnt_ref, m1t, b1, g1, bb1, m2t, b2, g2, bb2, z_o):
    pooled = sums_ref[...] / jnp.maximum(cnt_ref[...], 1.0)
    z = jnp.dot(pooled, m1t[...], preferred_element_type=jnp.float32) + b1[...]
    z = _layernorm(z, g1[...], bb1[...])
    z = jnp.maximum(z, 0.0)
    z = jnp.dot(z, m2t[...], preferred_element_type=jnp.float32) + b2[...]
    z_o[...] = _layernorm(z, g2[...], bb2[...])


def _row_spec():
    return pl.BlockSpec((RB, HC), lambda i: (i, 0))


def _head_spec():
    return pl.BlockSpec((RB, C), lambda i: (i, 0))


def _w_spec(r, c):
    return pl.BlockSpec((r, c), lambda i: (0, 0))


def _part_spec():
    return pl.BlockSpec((H, NC, RB, AW), lambda i: (0, 0, i, 0))


_F32 = jnp.float32


def _node_out():
    return jax.ShapeDtypeStruct((NP, HC), _F32)


def _head_out():
    return jax.ShapeDtypeStruct((NP, C), _F32)


_stage_a_call = pl.pallas_call(
    _stage_a,
    grid=(GRID,),
    in_specs=[_row_spec(), _w_spec(HC, HC), _w_spec(1, HC), _w_spec(HC, 512),
              _w_spec(1, 512), _w_spec(HC, HC), _w_spec(HC, HC)],
    out_specs=[_row_spec(), _row_spec()] + [_head_spec()] * (2 * H),
    out_shape=[_node_out(), _node_out()] + [_head_out()] * (2 * H),
)

_stage_b_call = pl.pallas_call(
    _stage_b,
    grid=(GRID,),
    in_specs=[_part_spec(), _w_spec(1, HC), _w_spec(1, HC), _w_spec(1, HC),
              _row_spec(), _row_spec(), _w_spec(HC, 512), _w_spec(HC, 512),
              _w_spec(1, 512), _w_spec(HC, HC), _w_spec(HC, HC)],
    out_specs=[_row_spec(), _row_spec()] + [_head_spec()] * (2 * H),
    out_shape=[_node_out(), _node_out()] + [_head_out()] * (2 * H),
)

_stage_c_call = pl.pallas_call(
    _stage_c,
    grid=(GRID,),
    in_specs=[_part_spec(), _w_spec(1, HC), _w_spec(1, HC), _w_spec(1, HC),
              _row_spec(), _row_spec(), _w_spec(HC, 512), _w_spec(HC, 512),
              _w_spec(1, 512), pl.BlockSpec((RB, 1), lambda i: (i, 0))],
    out_specs=[pl.BlockSpec((G, HC), lambda i: (0, 0)),
               pl.BlockSpec((G, HC), lambda i: (0, 0))],
    out_shape=[jax.ShapeDtypeStruct((G, HC), _F32),
               jax.ShapeDtypeStruct((G, HC), _F32)],
)

_final_call = pl.pallas_call(
    _final,
    out_shape=jax.ShapeDtypeStruct((G, HC), _F32),
)


def kernel(x, edge_index, batch, params):
    p = params
    e_in = edge_index.shape[1]
    loop = jnp.arange(N, dtype=jnp.int32)
    npad = EPAD - e_in - N
    src = jnp.concatenate([edge_index[0], loop,
                           jnp.zeros((npad,), jnp.int32)]).reshape(NW, CH, KE)
    dst = jnp.concatenate([edge_index[1], loop,
                           jnp.full((npad,), N, jnp.int32)]).reshape(NW, CH, KE)
    xp = jnp.pad(x, ((0, NP - N), (0, 0)))
    bpad = jnp.pad(batch, (0, NP - N), constant_values=G).reshape(NP, 1)

    def r1(v):
        return v.reshape(1, -1)

    bc0 = r1(p['rnn0_bih'] + p['rnn0_bhh'])
    h, c, *tabs = _stage_a_call(
        xp, p['lin0_W'].T, r1(p['lin0_b']), p['rnn0_Wih'].T, bc0,
        p['conv1_Wl'].T, p['conv1_Wr'].T)

    for i in (1, 2):
        part = _make_edge_kernel()(*tabs, src, dst, p['conv%d_att' % i].reshape(HC))
        bcomb = r1(p['conv%d_bih' % i] + p['conv%d_bhh' % i])
        h, c, *tabs = _stage_b_call(
            part, r1(p['conv%d_bias' % i]), r1(p['conv%d_ln_g' % i]),
            r1(p['conv%d_ln_b' % i]), h, c, p['conv%d_Wih' % i].T,
            p['conv%d_Whh' % i].T, bcomb,
            p['conv%d_Wl' % (i + 1)].T, p['conv%d_Wr' % (i + 1)].T)

    part = _make_edge_kernel()(*tabs, src, dst, p['conv3_att'].reshape(HC))
    bcomb = r1(p['conv3_bih'] + p['conv3_bhh'])
    sums, cnt = _stage_c_call(
        part, r1(p['conv3_bias']), r1(p['conv3_ln_g']), r1(p['conv3_ln_b']),
        h, c, p['conv3_Wih'].T, p['conv3_Whh'].T, bcomb, bpad)

    return _final_call(
        sums, cnt, p['mol1_W'].T, r1(p['mol1_b']), r1(p['ln1_g']),
        r1(p['ln1_b']), p['mol2_W'].T, r1(p['mol2_b']), r1(p['ln2_g']),
        r1(p['ln2_b']))


# speedup vs baseline: 1.0034x; 1.0034x over previous
import functools

import jax
import jax.numpy as jnp
from jax import lax
from jax.experimental import pallas as pl
from jax.experimental.pallas import tpu as pltpu
from jax.experimental.pallas import tpu_sc as plsc

N = 10000
NP = 10240
H = 4
C = 32
HC = 128
G = 64
AW = 48
NC, NS = 2, 16
NW = NC * NS
KE = 128
GP = 2
GE = GP * KE
CH = 84
NG = CH // GP
EPT = KE * CH
EPAD = EPT * NW
RB = 512
GRID = NP // RB
RPT = NP // NS
ZR = 64


def _edge_body(xl0, xl1, xl2, xl3, xr0, xr1, xr2, xr3, src_hbm, dst_hbm,
               att_hbm, out_hbm, src_v, dst_v, xl_a, xl_b, xr_a, xr_b,
               ctr_a, ctr_b, att_v, zrow, acc, sem_a, sem_b, sem_sa, sem_sb):
    cid = lax.axis_index("c")
    sid = lax.axis_index("s")
    wid = sid * NC + cid
    xls = (xl0, xl1, xl2, xl3)
    xrs = (xr0, xr1, xr2, xr3)

    def zr_init(i, carry):
        for j in range(AW // 16):
            zrow[i, pl.ds(16 * j, 16)] = jnp.zeros((16,), jnp.float32)
        return carry
    lax.fori_loop(0, ZR, zr_init, 0)

    def zero_acc(t, carry):
        pltpu.sync_copy(zrow, acc.at[pl.ds(sid * RPT + t * ZR, ZR)])
        return carry

    pltpu.sync_copy(src_hbm.at[wid], src_v)
    pltpu.sync_copy(dst_hbm.at[wid], dst_v)
    pltpu.sync_copy(att_hbm, att_v)
    lax.fori_loop(0, RPT // ZR, zero_acc, 0)
    plsc.subcore_barrier()

    lane = lax.iota(jnp.int32, 16)
    _dn = lax.GatherDimensionNumbers(offset_dims=(), collapsed_slice_dims=(0,),
                                     start_index_map=(0,))

    def _lanperm(vec, idx):
        return lax.gather(vec, idx.reshape(16, 1), _dn, (1,),
                          mode=lax.GatherScatterMode.PROMISE_IN_BOUNDS)

    for h in range(H):
        def fire_gather(g, xlb, xrb, sem, h=h):
            for j in range(GP):
                ch = GP * g + j
                pltpu.async_copy(xls[h].at[src_v.at[ch]],
                                 xlb.at[pl.ds(KE * j, KE)], sem)
                pltpu.async_copy(xrs[h].at[dst_v.at[ch]],
                                 xrb.at[pl.ds(KE * j, KE)], sem)

        def wait_gather(g, xlb, xrb, sem, h=h):
            for j in range(GP):
                ch = GP * g + j
                pltpu.make_async_copy(xls[h].at[src_v.at[ch]],
                                      xlb.at[pl.ds(KE * j, KE)], sem).wait()
                pltpu.make_async_copy(xrs[h].at[dst_v.at[ch]],
                                      xrb.at[pl.ds(KE * j, KE)], sem).wait()

        def fire_scatter(g, ctr, sem):
            for j in range(GP):
                pltpu.async_copy(ctr.at[pl.ds(KE * j, KE)],
                                 acc.at[dst_v.at[GP * g + j]], sem, add=True)

        def wait_scatter(g, ctr, sem):
            for j in range(GP):
                pltpu.make_async_copy(ctr.at[pl.ds(KE * j, KE)],
                                      acc.at[dst_v.at[GP * g + j]], sem).wait()

        def compute(xlb, xrb, ctr, h=h):
            att01 = (att_v[pl.ds(32 * h, 16)], att_v[pl.ds(32 * h + 16, 16)])

            @plsc.parallel_loop(0, GE, unroll=4, carry=att01)
            def _(e, att):
                at0, at1 = att
                a0 = xlb[e, pl.ds(0, 16)]
                a1 = xlb[e, pl.ds(16, 16)]
                b0 = xrb[e, pl.ds(0, 16)]
                b1 = xrb[e, pl.ds(16, 16)]
                v0 = a0 + b0
                v1 = a1 + b1
                l0 = jnp.where(v0 > 0, v0, 0.2 * v0)
                l1 = jnp.where(v1 > 0, v1, 0.2 * v1)
                t = l0 * at0 + l1 * at1
                for sh in (8, 4, 2, 1):
                    t = t + _lanperm(t, jnp.bitwise_xor(lane, sh))
                u = jnp.exp(t)
                ctr[e, pl.ds(0, 16)] = u * a0
                ctr[e, pl.ds(16, 16)] = u * a1
                ctr[e, pl.ds(32, 16)] = u
                return att

        fire_gather(0, xl_a, xr_a, sem_a)
        fire_gather(1, xl_b, xr_b, sem_b)
        wait_gather(0, xl_a, xr_a, sem_a)
        compute(xl_a, xr_a, ctr_a)
        fire_scatter(0, ctr_a, sem_sa)
        fire_gather(2, xl_a, xr_a, sem_a)
        wait_gather(1, xl_b, xr_b, sem_b)
        compute(xl_b, xr_b, ctr_b)
        fire_scatter(1, ctr_b, sem_sb)
        fire_gather(3, xl_b, xr_b, sem_b)

        def pair(k, carry):
            g0 = 2 * k + 2
            g1 = 2 * k + 3
            wait_gather(g0, xl_a, xr_a, sem_a)
            wait_scatter(g0 - 2, ctr_a, sem_sa)
            compute(xl_a, xr_a, ctr_a)
            fire_scatter(g0, ctr_a, sem_sa)
            fire_gather(g0 + 2, xl_a, xr_a, sem_a)
            wait_gather(g1, xl_b, xr_b, sem_b)
            wait_scatter(g1 - 2, ctr_b, sem_sb)
            compute(xl_b, xr_b, ctr_b)
            fire_scatter(g1, ctr_b, sem_sb)
            fire_gather(g1 + 2, xl_b, xr_b, sem_b)
            return carry

        lax.fori_loop(0, (NG - 4) // 2, pair, 0)

        g0 = NG - 2
        g1 = NG - 1
        wait_gather(g0, xl_a, xr_a, sem_a)
        wait_scatter(g0 - 2, ctr_a, sem_sa)
        compute(xl_a, xr_a, ctr_a)
        fire_scatter(g0, ctr_a, sem_sa)
        wait_gather(g1, xl_b, xr_b, sem_b)
        wait_scatter(g1 - 2, ctr_b, sem_sb)
        compute(xl_b, xr_b, ctr_b)
        fire_scatter(g1, ctr_b, sem_sb)
        wait_scatter(g0, ctr_a, sem_sa)
        wait_scatter(g1, ctr_b, sem_sb)

        plsc.subcore_barrier()
        pltpu.sync_copy(acc.at[pl.ds(sid * RPT, RPT)],
                        out_hbm.at[h, cid, pl.ds(sid * RPT, RPT)])
        lax.fori_loop(0, RPT // ZR, zero_acc, 0)
        plsc.subcore_barrier()


@functools.cache
def _make_edge_kernel():
  tab = jax.ShapeDtypeStruct((NP, C), jnp.float32)
  return pl.kernel(
    _edge_body,
    out_type=jax.ShapeDtypeStruct((H, NC, NP, AW), jnp.float32),
    mesh=plsc.VectorSubcoreMesh(core_axis_name="c", subcore_axis_name="s",
                                num_cores=NC, num_subcores=NS),
    compiler_params=pltpu.CompilerParams(use_tc_tiling_on_sc=False),
    scratch_types=[
        pltpu.VMEM((CH, KE), jnp.int32),
        pltpu.VMEM((CH, KE), jnp.int32),
        pltpu.VMEM((GE, C), jnp.float32),
        pltpu.VMEM((GE, C), jnp.float32),
        pltpu.VMEM((GE, C), jnp.float32),
        pltpu.VMEM((GE, C), jnp.float32),
        pltpu.VMEM((GE, AW), jnp.float32),
        pltpu.VMEM((GE, AW), jnp.float32),
        pltpu.VMEM((HC,), jnp.float32),
        pltpu.VMEM((ZR, AW), jnp.float32),
        pltpu.VMEM_SHARED((NP, AW), jnp.float32),
        pltpu.SemaphoreType.DMA,
        pltpu.SemaphoreType.DMA,
        pltpu.SemaphoreType.DMA,
        pltpu.SemaphoreType.DMA,
    ],
  )


def _lstm(y, h_prev, c_prev, wiht, whht, bcomb):
    gates = (jnp.dot(y, wiht, preferred_element_type=jnp.float32)
             + jnp.dot(h_prev, whht, preferred_element_type=jnp.float32)
             + bcomb)
    i = jax.nn.sigmoid(gates[:, 0:128])
    f = jax.nn.sigmoid(gates[:, 128:256])
    g = jnp.tanh(gates[:, 256:384])
    o = jax.nn.sigmoid(gates[:, 384:512])
    c2 = f * c_prev + i * g
    h2 = o * jnp.tanh(c2)
    return h2, c2


def _layernorm(y, g, b):
    m = jnp.mean(y, axis=-1, keepdims=True)
    v = jnp.mean((y - m) * (y - m), axis=-1, keepdims=True)
    return (y - m) / jnp.sqrt(v + 1e-5) * g + b


def _norm_ln_elu(part_ref, bias, ln_g, ln_b):
    ys = []
    for h in range(H):
        a = part_ref[h, 0] + part_ref[h, 1]
        den = jnp.maximum(a[:, 32:33], 1e-16)
        ys.append(a[:, 0:32] / den)
    y = jnp.concatenate(ys, axis=1) + bias
    y = _layernorm(y, ln_g, ln_b)
    return jnp.where(y > 0, y, jnp.exp(y) - 1.0)


def _split_heads(xfull, outs):
    for j in range(H):
        outs[j][...] = xfull[:, 32 * j:32 * j + 32]


def _stage_a(x_ref, w0t, b0, wih0t, bc0, wlt, wrt, h_o, c_o, *lr_o):
    xb = x_ref[...]
    x0 = jnp.dot(xb, w0t[...], preferred_element_type=jnp.float32) + b0[...]
    gates = jnp.dot(x0, wih0t[...], preferred_element_type=jnp.float32) + bc0[...]
    i = jax.nn.sigmoid(gates[:, 0:128])
    g = jnp.tanh(gates[:, 256:384])
    o = jax.nn.sigmoid(gates[:, 384:512])
    c2 = i * g
    h_o[...] = o * jnp.tanh(c2)
    c_o[...] = c2
    _split_heads(jnp.dot(xb, wlt[...], preferred_element_type=jnp.float32),
                 lr_o[0:H])
    _split_heads(jnp.dot(xb, wrt[...], preferred_element_type=jnp.float32),
                 lr_o[H:2 * H])


def _stage_b(part_ref, bias, ln_g, ln_b, h_ref, c_ref, wiht, whht, bcomb,
             wlt, wrt, h_o, c_o, *lr_o):
    y = _norm_ln_elu(part_ref, bias[...], ln_g[...], ln_b[...])
    h2, c2 = _lstm(y, h_ref[...], c_ref[...], wiht[...], whht[...], bcomb[...])
    h_o[...] = h2
    c_o[...] = c2
    _split_heads(jnp.dot(h2, wlt[...], preferred_element_type=jnp.float32),
                 lr_o[0:H])
    _split_heads(jnp.dot(h2, wrt[...], preferred_element_type=jnp.float32),
                 lr_o[H:2 * H])


def _stage_c(part_ref, bias, ln_g, ln_b, h_ref, c_ref, wiht, whht, bcomb,
             batch_ref, sums_o, cnt_o):
    pid = pl.program_id(0)

    @pl.when(pid == 0)
    def _():
        sums_o[...] = jnp.zeros_like(sums_o)
        cnt_o[...] = jnp.zeros_like(cnt_o)

    y = _norm_ln_elu(part_ref, bias[...], ln_g[...], ln_b[...])
    h2, _ = _lstm(y, h_ref[...], c_ref[...], wiht[...], whht[...], bcomb[...])
    onehot = (batch_ref[...] == lax.broadcasted_iota(jnp.int32, (1, G), 1))
    onehot = onehot.astype(jnp.float32)
    sums_o[...] += lax.dot_general(onehot, h2, (((0,), (0,)), ((), ())),
                                   preferred_element_type=jnp.float32)
    cnt = jnp.sum(onehot, axis=0)
    cnt_o[...] += jnp.broadcast_to(cnt.reshape(G, 1), (G, HC))


def _final(sums_ref, cnt_ref, m1t, b1, g1, bb1, m2t, b2, g2, bb2, z_o):
    pooled = sums_ref[...] / jnp.maximum(cnt_ref[...], 1.0)
    z = jnp.dot(pooled, m1t[...], preferred_element_type=jnp.float32) + b1[...]
    z = _layernorm(z, g1[...], bb1[...])
    z = jnp.maximum(z, 0.0)
    z = jnp.dot(z, m2t[...], preferred_element_type=jnp.float32) + b2[...]
    z_o[...] = _layernorm(z, g2[...], bb2[...])


def _row_spec():
    return pl.BlockSpec((RB, HC), lambda i: (i, 0))


def _head_spec():
    return pl.BlockSpec((RB, C), lambda i: (i, 0))


def _w_spec(r, c):
    return pl.BlockSpec((r, c), lambda i: (0, 0))


def _part_spec():
    return pl.BlockSpec((H, NC, RB, AW), lambda i: (0, 0, i, 0))


_F32 = jnp.float32


def _node_out():
    return jax.ShapeDtypeStruct((NP, HC), _F32)


def _head_out():
    return jax.ShapeDtypeStruct((NP, C), _F32)


_stage_a_call = pl.pallas_call(
    _stage_a,
    grid=(GRID,),
    in_specs=[_row_spec(), _w_spec(HC, HC), _w_spec(1, HC), _w_spec(HC, 512),
              _w_spec(1, 512), _w_spec(HC, HC), _w_spec(HC, HC)],
    out_specs=[_row_spec(), _row_spec()] + [_head_spec()] * (2 * H),
    out_shape=[_node_out(), _node_out()] + [_head_out()] * (2 * H),
)

_stage_b_call = pl.pallas_call(
    _stage_b,
    grid=(GRID,),
    in_specs=[_part_spec(), _w_spec(1, HC), _w_spec(1, HC), _w_spec(1, HC),
              _row_spec(), _row_spec(), _w_spec(HC, 512), _w_spec(HC, 512),
              _w_spec(1, 512), _w_spec(HC, HC), _w_spec(HC, HC)],
    out_specs=[_row_spec(), _row_spec()] + [_head_spec()] * (2 * H),
    out_shape=[_node_out(), _node_out()] + [_head_out()] * (2 * H),
)

_stage_c_call = pl.pallas_call(
    _stage_c,
    grid=(GRID,),
    in_specs=[_part_spec(), _w_spec(1, HC), _w_spec(1, HC), _w_spec(1, HC),
              _row_spec(), _row_spec(), _w_spec(HC, 512), _w_spec(HC, 512),
              _w_spec(1, 512), pl.BlockSpec((RB, 1), lambda i: (i, 0))],
    out_specs=[pl.BlockSpec((G, HC), lambda i: (0, 0)),
               pl.BlockSpec((G, HC), lambda i: (0, 0))],
    out_shape=[jax.ShapeDtypeStruct((G, HC), _F32),
               jax.ShapeDtypeStruct((G, HC), _F32)],
)

_final_call = pl.pallas_call(
    _final,
    out_shape=jax.ShapeDtypeStruct((G, HC), _F32),
)


def kernel(x, edge_index, batch, params):
    p = params
    e_in = edge_index.shape[1]
    loop = jnp.arange(N, dtype=jnp.int32)
    npad = EPAD - e_in - N
    src = jnp.concatenate([edge_index[0], loop,
                           jnp.zeros((npad,), jnp.int32)]).reshape(NW, CH, KE)
    dst = jnp.concatenate([edge_index[1], loop,
                           jnp.full((npad,), N, jnp.int32)]).reshape(NW, CH, KE)
    xp = jnp.pad(x, ((0, NP - N), (0, 0)))
    bpad = jnp.pad(batch, (0, NP - N), constant_values=G).reshape(NP, 1)

    def r1(v):
        return v.reshape(1, -1)

    bc0 = r1(p['rnn0_bih'] + p['rnn0_bhh'])
    h, c, *tabs = _stage_a_call(
        xp, p['lin0_W'].T, r1(p['lin0_b']), p['rnn0_Wih'].T, bc0,
        p['conv1_Wl'].T, p['conv1_Wr'].T)

    for i in (1, 2):
        part = _make_edge_kernel()(*tabs, src, dst, p['conv%d_att' % i].reshape(HC))
        bcomb = r1(p['conv%d_bih' % i] + p['conv%d_bhh' % i])
        h, c, *tabs = _stage_b_call(
            part, r1(p['conv%d_bias' % i]), r1(p['conv%d_ln_g' % i]),
            r1(p['conv%d_ln_b' % i]), h, c, p['conv%d_Wih' % i].T,
            p['conv%d_Whh' % i].T, bcomb,
            p['conv%d_Wl' % (i + 1)].T, p['conv%d_Wr' % (i + 1)].T)

    part = _make_edge_kernel()(*tabs, src, dst, p['conv3_att'].reshape(HC))
    bcomb = r1(p['conv3_bih'] + p['conv3_bhh'])
    sums, cnt = _stage_c_call(
        part, r1(p['conv3_bias']), r1(p['conv3_ln_g']), r1(p['conv3_ln_b']),
        h, c, p['conv3_Wih'].T, p['conv3_Whh'].T, bcomb, bpad)

    return _final_call(
        sums, cnt, p['mol1_W'].T, r1(p['mol1_b']), r1(p['ln1_g']),
        r1(p['ln1_b']), p['mol2_W'].T, r1(p['mol2_b']), r1(p['ln2_g']),
        r1(p['ln2_b']))
